# baseline (speedup 1.0000x reference)
import functools

import jax
import jax.numpy as jnp
from jax.experimental import pallas as pl
from jax.experimental.pallas import tpu as pltpu

B = 4096
IN = 2048
H1 = 512
H2 = 256
P = 16
K = 128
D = 64

BLK = 512
GRID = B // BLK


def _fused_body(di_ref, x_ref, w1_ref, b1_ref, w2_ref, b2_ref, wpf_ref,
                bpf_ref, tab_ref, out_ref):
    x = x_ref[...]
    h = jnp.maximum(jnp.dot(x, w1_ref[...],
                            preferred_element_type=jnp.float32)
                    + b1_ref[0, :], 0.0)
    h = jnp.maximum(jnp.dot(h.astype(jnp.bfloat16), w2_ref[...],
                            preferred_element_type=jnp.float32)
                    + b2_ref[0, :], 0.0)
    a = jnp.maximum(jnp.dot(h.astype(jnp.bfloat16), wpf_ref[...],
                            preferred_element_type=jnp.float32)
                    + bpf_ref[0, :], 0.0)

    di = di_ref[0, 0, :]
    onehot = (di[:, None] ==
              jax.lax.broadcasted_iota(jnp.int32, (BLK, D), 1)
              ).astype(jnp.float32)
    g = jnp.dot(onehot, tab_ref[...], preferred_element_type=jnp.float32)
    wdg = g[:, :K]
    bdg = g[:, K]
    pwf = g[:, K + 1]

    acc = jnp.zeros((BLK,), dtype=jnp.float32)
    for p in range(P):
        cp = jnp.sum(a[:, p * K:(p + 1) * K] * wdg, axis=1)
        acc = jnp.where(pwf == float(p), cp, acc)
    out_ref[0, 0, :] = acc + bdg


def kernel(x, drug_indices, drug_to_pw, W1, b1, W2, b2, Wp, bp, Wd, bd):
    xb = x.astype(jnp.bfloat16)
    w1b = W1.astype(jnp.bfloat16)
    w2b = W2.astype(jnp.bfloat16)
    wpf = Wp.transpose(1, 0, 2).reshape(H2, P * K).astype(jnp.bfloat16)
    bpf = bp.reshape(1, P * K)
    tab = jnp.concatenate(
        [Wd, bd[:, None], drug_to_pw.astype(jnp.float32)[:, None]], axis=1)
    di3 = drug_indices.reshape(GRID, 1, BLK)

    out = pl.pallas_call(
        _fused_body,
        grid=(GRID,),
        in_specs=[
            pl.BlockSpec((1, 1, BLK), lambda i: (i, 0, 0)),
            pl.BlockSpec((BLK, IN), lambda i: (i, 0)),
            pl.BlockSpec((IN, H1), lambda i: (0, 0)),
            pl.BlockSpec((1, H1), lambda i: (0, 0)),
            pl.BlockSpec((H1, H2), lambda i: (0, 0)),
            pl.BlockSpec((1, H2), lambda i: (0, 0)),
            pl.BlockSpec((H2, P * K), lambda i: (0, 0)),
            pl.BlockSpec((1, P * K), lambda i: (0, 0)),
            pl.BlockSpec((D, K + 2), lambda i: (0, 0)),
        ],
        out_specs=pl.BlockSpec((1, 1, BLK), lambda i: (i, 0, 0)),
        out_shape=jax.ShapeDtypeStruct((GRID, 1, BLK), jnp.float32),
    )(di3, xb, w1b, b1.reshape(1, H1), w2b, b2.reshape(1, H2), wpf, bpf,
      tab)
    return out.reshape(B)


# speedup vs baseline: 1.7735x; 1.7735x over previous
import functools

import jax
import jax.numpy as jnp
from jax.experimental import pallas as pl
from jax.experimental.pallas import tpu as pltpu

B = 4096
IN = 2048
H1 = 512
H2 = 256
P = 16
K = 128
D = 64

BLK = 512
GRID = B // BLK


def _fused_body(di_ref, x_ref, w1_ref, b1_ref, w2_ref, b2_ref, wpf_ref,
                bpf_ref, tab_ref, out_ref):
    x = x_ref[...].astype(jnp.bfloat16)
    h = jnp.maximum(jnp.dot(x, w1_ref[...].astype(jnp.bfloat16),
                            preferred_element_type=jnp.float32)
                    + b1_ref[0, :], 0.0)
    h = jnp.maximum(jnp.dot(h.astype(jnp.bfloat16),
                            w2_ref[...].astype(jnp.bfloat16),
                            preferred_element_type=jnp.float32)
                    + b2_ref[0, :], 0.0)
    a = jnp.maximum(jnp.dot(h.astype(jnp.bfloat16),
                            wpf_ref[...].astype(jnp.bfloat16),
                            preferred_element_type=jnp.float32)
                    + bpf_ref[0, :], 0.0)

    di = di_ref[0, 0, :]
    onehot = (di[:, None] ==
              jax.lax.broadcasted_iota(jnp.int32, (BLK, D), 1)
              ).astype(jnp.float32)
    g = jnp.dot(onehot, tab_ref[...], preferred_element_type=jnp.float32)
    wdg = g[:, :K]
    bdg = g[:, K]
    pwf = g[:, K + 1]

    sel = a[:, :K]
    for p in range(1, P):
        sel = jnp.where((pwf == float(p))[:, None],
                        a[:, p * K:(p + 1) * K], sel)
    out_ref[0, 0, :] = jnp.sum(sel * wdg, axis=1) + bdg


def kernel(x, drug_indices, drug_to_pw, W1, b1, W2, b2, Wp, bp, Wd, bd):
    wpf = Wp.transpose(1, 0, 2).reshape(H2, P * K)
    bpf = bp.reshape(1, P * K)
    tab = jnp.concatenate(
        [Wd, bd[:, None], drug_to_pw.astype(jnp.float32)[:, None]], axis=1)
    di3 = drug_indices.reshape(GRID, 1, BLK)

    out = pl.pallas_call(
        _fused_body,
        grid=(GRID,),
        in_specs=[
            pl.BlockSpec((1, 1, BLK), lambda i: (i, 0, 0)),
            pl.BlockSpec((BLK, IN), lambda i: (i, 0)),
            pl.BlockSpec((IN, H1), lambda i: (0, 0)),
            pl.BlockSpec((1, H1), lambda i: (0, 0)),
            pl.BlockSpec((H1, H2), lambda i: (0, 0)),
            pl.BlockSpec((1, H2), lambda i: (0, 0)),
            pl.BlockSpec((H2, P * K), lambda i: (0, 0)),
            pl.BlockSpec((1, P * K), lambda i: (0, 0)),
            pl.BlockSpec((D, K + 2), lambda i: (0, 0)),
        ],
        out_specs=pl.BlockSpec((1, 1, BLK), lambda i: (i, 0, 0)),
        out_shape=jax.ShapeDtypeStruct((GRID, 1, BLK), jnp.float32),
    )(di3, x, W1, b1.reshape(1, H1), W2, b2.reshape(1, H2), wpf, bpf, tab)
    return out.reshape(B)
